# baseline (speedup 1.0000x reference)
import jax
import jax.numpy as jnp
from jax.experimental import pallas as pl


def _sage_fused(adj_ref, x_ref,
                wl0_ref, bl0_ref, wr0_ref,
                wl1_ref, bl1_ref, wr1_ref,
                wl2_ref, bl2_ref, wr2_ref,
                out_ref):
    adj = adj_ref[...]
    g = jnp.transpose(x_ref[...])

    deg = jnp.sum(adj, axis=0, keepdims=True, dtype=jnp.float32)
    dinv = 1.0 / jnp.maximum(deg, 1.0)

    def layer(gh, wl_ref, bl_ref, wr_ref):
        aggT = jnp.dot(gh.astype(jnp.bfloat16), adj,
                       preferred_element_type=jnp.float32)
        meanT = aggT * dinv
        lin_l = jnp.dot(jnp.transpose(wl_ref[...]), meanT,
                        preferred_element_type=jnp.float32)
        lin_r = jnp.dot(jnp.transpose(wr_ref[...]), gh,
                        preferred_element_type=jnp.float32)
        return lin_l + lin_r + jnp.transpose(bl_ref[...])

    g = jnp.maximum(layer(g, wl0_ref, bl0_ref, wr0_ref), 0.0)
    g = jnp.maximum(layer(g, wl1_ref, bl1_ref, wr1_ref), 0.0)
    out_ref[...] = jnp.transpose(layer(g, wl2_ref, bl2_ref, wr2_ref))


def kernel(x, adj, W_l0, b_l0, W_r0, W_l1, b_l1, W_r1, W_l2, b_l2, W_r2):
    n, _ = x.shape
    d_out = W_l2.shape[1]
    return pl.pallas_call(
        _sage_fused,
        out_shape=jax.ShapeDtypeStruct((n, d_out), jnp.float32),
    )(adj.astype(jnp.bfloat16), x,
      W_l0, b_l0.reshape(1, -1), W_r0,
      W_l1, b_l1.reshape(1, -1), W_r1,
      W_l2, b_l2.reshape(1, -1), W_r2)


# speedup vs baseline: 1.3773x; 1.3773x over previous
import jax
import jax.numpy as jnp
from jax.experimental import pallas as pl
from jax.experimental.pallas import tpu as pltpu

_NB = 6


def _sage_fused(adj_hbm, x_ref,
                wl0_ref, bl0_ref, wr0_ref,
                wl1_ref, bl1_ref, wr1_ref,
                wl2_ref, bl2_ref, wr2_ref,
                out_ref, adj_vmem, sems):
    n = adj_vmem.shape[0]
    w = n // _NB
    copies = [
        pltpu.make_async_copy(adj_hbm.at[pl.ds(b * w, w), :],
                              adj_vmem.at[pl.ds(b * w, w), :],
                              sems.at[b])
        for b in range(_NB)
    ]
    for c in copies:
        c.start()
    g = jnp.transpose(x_ref[...])

    aggT = None
    deg = None
    for b in range(_NB):
        copies[b].wait()
        blk = adj_vmem[pl.ds(b * w, w), :]
        part = jnp.dot(g[:, b * w:(b + 1) * w], blk,
                       preferred_element_type=jnp.float32)
        degp = jnp.sum(blk, axis=0, keepdims=True)
        aggT = part if aggT is None else aggT + part
        deg = degp if deg is None else deg + degp
    dinv = 1.0 / jnp.maximum(deg, 1.0)

    def tail(gh, aggT_l, wl_ref, bl_ref, wr_ref):
        meanT = aggT_l * dinv
        lin_l = jnp.dot(jnp.transpose(wl_ref[...]), meanT,
                        preferred_element_type=jnp.float32)
        lin_r = jnp.dot(jnp.transpose(wr_ref[...]), gh,
                        preferred_element_type=jnp.float32)
        return lin_l + lin_r + jnp.transpose(bl_ref[...])

    adj = adj_vmem[...]
    g = jnp.maximum(tail(g, aggT, wl0_ref, bl0_ref, wr0_ref), 0.0)
    aggT = jnp.dot(g, adj, preferred_element_type=jnp.float32)
    g = jnp.maximum(tail(g, aggT, wl1_ref, bl1_ref, wr1_ref), 0.0)
    aggT = jnp.dot(g, adj, preferred_element_type=jnp.float32)
    out_ref[...] = jnp.transpose(tail(g, aggT, wl2_ref, bl2_ref, wr2_ref))


def kernel(x, adj, W_l0, b_l0, W_r0, W_l1, b_l1, W_r1, W_l2, b_l2, W_r2):
    n, _ = x.shape
    d_out = W_l2.shape[1]
    return pl.pallas_call(
        _sage_fused,
        out_shape=jax.ShapeDtypeStruct((n, d_out), jnp.float32),
        in_specs=[pl.BlockSpec(memory_space=pl.ANY)]
        + [pl.BlockSpec(memory_space=pltpu.VMEM)] * 10,
        scratch_shapes=[pltpu.VMEM((n, n), jnp.float32),
                        pltpu.SemaphoreType.DMA((_NB,))],
    )(adj, x,
      W_l0, b_l0.reshape(1, -1), W_r0,
      W_l1, b_l1.reshape(1, -1), W_r1,
      W_l2, b_l2.reshape(1, -1), W_r2)


# speedup vs baseline: 1.4071x; 1.0216x over previous
import jax
import jax.numpy as jnp
from jax.experimental import pallas as pl


def _sage_fused(adj_ref, x_ref,
                wl0_ref, bl0_ref, wr0_ref,
                wl1_ref, bl1_ref, wr1_ref,
                wl2_ref, bl2_ref, wr2_ref,
                out_ref):
    adj = adj_ref[...]
    n = adj.shape[0]
    gx = jnp.transpose(x_ref[...])
    d = gx.shape[0]

    g0 = jnp.concatenate([gx, jnp.ones((1, n), jnp.float32)], axis=0)
    first = jnp.dot(g0, adj, preferred_element_type=jnp.float32)
    dinv = 1.0 / jnp.maximum(first[d:, :], 1.0)

    def tail(gh, aggT, wl_ref, bl_ref, wr_ref):
        rhs = jnp.concatenate([aggT * dinv, gh], axis=0)
        w = jnp.concatenate([jnp.transpose(wl_ref[...]),
                             jnp.transpose(wr_ref[...])], axis=1)
        return jnp.dot(w, rhs, preferred_element_type=jnp.float32) \
            + jnp.transpose(bl_ref[...])

    g = jnp.maximum(tail(gx, first[:d, :], wl0_ref, bl0_ref, wr0_ref), 0.0)
    aggT = jnp.dot(g, adj, preferred_element_type=jnp.float32)
    g = jnp.maximum(tail(g, aggT, wl1_ref, bl1_ref, wr1_ref), 0.0)
    aggT = jnp.dot(g, adj, preferred_element_type=jnp.float32)
    out_ref[...] = jnp.transpose(tail(g, aggT, wl2_ref, bl2_ref, wr2_ref))


def kernel(x, adj, W_l0, b_l0, W_r0, W_l1, b_l1, W_r1, W_l2, b_l2, W_r2):
    n, _ = x.shape
    d_out = W_l2.shape[1]
    return pl.pallas_call(
        _sage_fused,
        out_shape=jax.ShapeDtypeStruct((n, d_out), jnp.float32),
    )(adj, x,
      W_l0, b_l0.reshape(1, -1), W_r0,
      W_l1, b_l1.reshape(1, -1), W_r1,
      W_l2, b_l2.reshape(1, -1), W_r2)
